# baseline (speedup 1.0000x reference)
import jax
import jax.numpy as jnp
from jax import lax
from jax.experimental import pallas as pl
from jax.experimental.pallas import tpu as pltpu
from jax.experimental.pallas import tpu_sc as plsc

N = 16384
D = 64
C = 100000
DP = 2 * D
CP = C // 2
NP = N // 2
NC = 2
NS = 16
NW = NC * NS
CPT = 3136
CPT_LAST = C - (NW - 1) * CPT
PAIRS = CPT // 2
PAIRS_LAST = CPT_LAST // 2
SLOTS = 512
LCHUNK = 1024
GB = 128
CPROW = 128
LIST_CAP = N + 16


def _body(features, labels, centers, out,
          lab_buf, list_idx, list_crel, counts, slotmap, sums,
          wk_idx, wk_slot, wk_pair, bl_pair, scat_idx, rows,
          cp0, cp1, sem_g, sem_r0, sem_r1, sem_w0, sem_w1):
  wid = lax.axis_index("s") * NC + lax.axis_index("c")
  lo = wid * CPT
  lo_pair = wid * PAIRS
  iota16 = lax.iota(jnp.int32, 16)
  zeros16f = jnp.zeros((16,), jnp.float32)
  zeros16i = jnp.zeros((16,), jnp.int32)

  def compress_store(ref, x, m, base):
    mi = m.astype(jnp.int32)
    dest = base + plsc.cumsum(mi) - mi
    plsc.store_scatter(ref, [dest], x, mask=m)
    return plsc.all_reduce_population_count(m)[0]

  def z_list(i, _):
    list_idx[pl.ds(i * 16, 16)] = zeros16i
    list_crel[pl.ds(i * 16, 16)] = zeros16i
    return 0
  lax.fori_loop(0, LIST_CAP // 16, z_list, 0)
  def z_small(i, _):
    wk_idx[pl.ds(i * 16, 16)] = zeros16i
    wk_slot[pl.ds(i * 16, 16)] = zeros16i
    wk_pair[pl.ds(i * 16, 16)] = zeros16i
    bl_pair[pl.ds(i * 16, 16)] = zeros16i
    return 0
  lax.fori_loop(0, (GB + 32) // 16, z_small, 0)

  def build_chunk(chunk, cur):
    pltpu.sync_copy(labels.at[pl.ds(chunk * LCHUNK, LCHUNK)], lab_buf)
    def group(g, cur):
      lab = lab_buf[pl.ds(g * 16, 16)]
      crel = lab - lo
      pos = iota16 + (chunk * LCHUNK + g * 16)
      m = (crel >= 0) & (crel < CPT)
      compress_store(list_idx, pos, m, cur)
      return cur + compress_store(list_crel, crel, m, cur)
    return lax.fori_loop(0, LCHUNK // 16, group, cur)
  cur = lax.fori_loop(0, N // LCHUNK, build_chunk, jnp.int32(0))

  def z_counts(z, _):
    counts[pl.ds(z * 16, 16)] = zeros16i
    return 0
  lax.fori_loop(0, CPT // 16, z_counts, 0)

  one_hot0 = (iota16 == 0).astype(jnp.int32)
  def add_count(i, _):
    c = list_crel[pl.ds(i, 16)][0]
    cnts = counts[pl.ds(c, 16)]
    counts[pl.ds(c, 16)] = cnts + one_hot0
    return 0
  lax.fori_loop(0, cur, add_count, 0)

  def slot_group(z, base):
    cv = counts[pl.ds(z * 16, 16)]
    mb = cv > 0
    m = mb.astype(jnp.int32)
    incl = plsc.cumsum(m)
    slotmap[pl.ds(z * 16, 16)] = base + incl - m
    return base + plsc.all_reduce_population_count(mb)[0]
  total_touched = lax.fori_loop(0, CPT // 16, slot_group, jnp.int32(0))

  bufs = [cp0, cp1]
  rsems = [sem_r0, sem_r1]
  wsems = [sem_w0, sem_w1]

  def copy_range(nrows):
    nfull = nrows // CPROW
    rem = nrows - nfull * CPROW
    chunks = [(i * CPROW, CPROW) for i in range(nfull)]
    if rem:
      chunks.append((nfull * CPROW, rem))
    pend_w = [None, None]
    pend_r = [None, None]
    st0, sz0 = chunks[0]
    pend_r[0] = pltpu.async_copy(
        centers.at[pl.ds(lo_pair + st0, sz0)], bufs[0].at[pl.ds(0, sz0)],
        rsems[0])
    for i, (st, sz) in enumerate(chunks):
      b = i % 2
      pend_r[b].wait()
      if i + 1 < len(chunks):
        nb = (i + 1) % 2
        if pend_w[nb] is not None:
          pend_w[nb].wait()
          pend_w[nb] = None
        nst, nsz = chunks[i + 1]
        pend_r[nb] = pltpu.async_copy(
            centers.at[pl.ds(lo_pair + nst, nsz)], bufs[nb].at[pl.ds(0, nsz)],
            rsems[nb])
      pend_w[b] = pltpu.async_copy(
          bufs[b].at[pl.ds(0, sz)], out.at[pl.ds(lo_pair + st, sz)], wsems[b])
    for b in range(2):
      if pend_w[b] is not None:
        pend_w[b].wait()

  @pl.when(wid < NW - 1)
  def _():
    copy_range(PAIRS)

  @pl.when(wid == NW - 1)
  def _():
    copy_range(PAIRS_LAST)

  def do_accum_flush(nvalid):
    pltpu.async_copy(features.at[wk_pair.at[pl.ds(0, GB)]], rows, sem_g).wait()
    def acc_row(j, _):
      @pl.when(j < nvalid)
      def _():
        s = wk_slot[pl.ds(j, 16)][0]
        par = wk_idx[pl.ds(j, 16)][0] & 1
        col0 = par * D
        for kk in range(D // 16):
          sv = sums[pl.ds(s * D + kk * 16, 16)]
          rv = rows[j, pl.ds(col0 + kk * 16, 16)]
          sums[pl.ds(s * D + kk * 16, 16)] = sv + rv
      return 0
    lax.fori_loop(0, GB, acc_row, 0)

  def do_blend_flush(nvalid, slot_lo):
    for zz in range(GB // 16):
      scat_idx[pl.ds(zz * 16, 16)] = bl_pair[pl.ds(zz * 16, 16)] + lo_pair
    pltpu.async_copy(out.at[scat_idx], rows, sem_g).wait()
    def blend_row(j, _):
      @pl.when(j < nvalid)
      def _():
        p = bl_pair[pl.ds(j, 16)][0]
        for h in range(2):
          c = 2 * p + h
          cnt = counts[pl.ds(c, 16)][0]
          s = slotmap[pl.ds(c, 16)][0]
          @pl.when((cnt > 0) & (s >= slot_lo) & (s < slot_lo + SLOTS))
          def _():
            srel = s - slot_lo
            cnt_v = jnp.full((16,), cnt, jnp.int32).astype(jnp.float32)
            w = 0.5 / cnt_v
            for kk in range(D // 16):
              col = h * D + kk * 16
              cvv = rows[j, pl.ds(col, 16)]
              svv = sums[pl.ds(srel * D + kk * 16, 16)]
              rows[j, pl.ds(col, 16)] = 0.5 * cvv + w * svv
      return 0
    lax.fori_loop(0, GB, blend_row, 0)
    s0 = scat_idx[pl.ds(0, 16)][0]
    for zz in range(GB // 16):
      lane_pos = iota16 + zz * 16
      curv = scat_idx[pl.ds(zz * 16, 16)]
      scat_idx[pl.ds(zz * 16, 16)] = jnp.where(lane_pos >= nvalid, s0, curv)
    row0s = [rows[0, pl.ds(kk * 16, 16)] for kk in range(DP // 16)]
    def tail_row(j, _):
      for kk in range(DP // 16):
        rows[j, pl.ds(kk * 16, 16)] = row0s[kk]
      return 0
    lax.fori_loop(nvalid, GB, tail_row, 0)
    pltpu.async_copy(rows, out.at[scat_idx], sem_g).wait()

  def chunk_pass(k, _):
    slot_lo = k * SLOTS
    def z_sums(i, _):
      for kk in range(4):
        sums[pl.ds(i * 64 + kk * 16, 16)] = zeros16f
      return 0
    lax.fori_loop(0, SLOTS * D // 64, z_sums, 0)

    ngroups = (cur + 15) // 16
    def agroup(g, wcur):
      crel16 = list_crel[pl.ds(g * 16, 16)]
      idx16 = list_idx[pl.ds(g * 16, 16)]
      pos = iota16 + g * 16
      slot16 = plsc.load_gather(slotmap, [crel16])
      m = (pos < cur) & (slot16 >= slot_lo) & (slot16 < slot_lo + SLOTS)
      compress_store(wk_idx, idx16, m, wcur)
      compress_store(wk_pair, idx16 >> 1, m, wcur)
      wcur = wcur + compress_store(wk_slot, slot16 - slot_lo, m, wcur)
      @pl.when(wcur >= GB)
      def _():
        do_accum_flush(GB)
        wk_idx[pl.ds(0, 16)] = wk_idx[pl.ds(GB, 16)]
        wk_pair[pl.ds(0, 16)] = wk_pair[pl.ds(GB, 16)]
        wk_slot[pl.ds(0, 16)] = wk_slot[pl.ds(GB, 16)]
      return lax.select(wcur >= GB, wcur - GB, wcur)
    wcur = lax.fori_loop(0, ngroups, agroup, jnp.int32(0))
    @pl.when(wcur > 0)
    def _():
      do_accum_flush(wcur)

    def bgroup(z, bcur):
      p16 = iota16 + z * 16
      ce = plsc.load_gather(counts, [p16 * 2])
      co = plsc.load_gather(counts, [p16 * 2 + 1])
      se = plsc.load_gather(slotmap, [p16 * 2])
      so = plsc.load_gather(slotmap, [p16 * 2 + 1])
      me = (ce > 0) & (se >= slot_lo) & (se < slot_lo + SLOTS)
      mo = (co > 0) & (so >= slot_lo) & (so < slot_lo + SLOTS)
      m = me | mo
      bcur = bcur + compress_store(bl_pair, p16, m, bcur)
      @pl.when(bcur >= GB)
      def _():
        do_blend_flush(GB, slot_lo)
        bl_pair[pl.ds(0, 16)] = bl_pair[pl.ds(GB, 16)]
      return lax.select(bcur >= GB, bcur - GB, bcur)
    bcur = lax.fori_loop(0, PAIRS // 16, bgroup, jnp.int32(0))
    @pl.when(bcur > 0)
    def _():
      do_blend_flush(bcur, slot_lo)
    return 0

  nchunks = (total_touched + SLOTS - 1) // SLOTS
  lax.fori_loop(0, nchunks, chunk_pass, 0)


@jax.jit
def _run(features, labels, centers):
  mesh = plsc.VectorSubcoreMesh(core_axis_name="c", subcore_axis_name="s",
                                num_cores=NC, num_subcores=NS)
  kern = pl.kernel(
      _body,
      out_type=jax.ShapeDtypeStruct((CP, DP), jnp.float32),
      mesh=mesh,
      compiler_params=pltpu.CompilerParams(needs_layout_passes=False),
      scratch_types=[
          pltpu.VMEM((LCHUNK,), jnp.int32),
          pltpu.VMEM((LIST_CAP,), jnp.int32),
          pltpu.VMEM((LIST_CAP,), jnp.int32),
          pltpu.VMEM((CPT + 16,), jnp.int32),
          pltpu.VMEM((CPT + 16,), jnp.int32),
          pltpu.VMEM((SLOTS * D,), jnp.float32),
          pltpu.VMEM((GB + 32,), jnp.int32),
          pltpu.VMEM((GB + 32,), jnp.int32),
          pltpu.VMEM((GB + 32,), jnp.int32),
          pltpu.VMEM((GB + 32,), jnp.int32),
          pltpu.VMEM((GB,), jnp.int32),
          pltpu.VMEM((GB, DP), jnp.float32),
          pltpu.VMEM((CPROW, DP), jnp.float32),
          pltpu.VMEM((CPROW, DP), jnp.float32),
          pltpu.SemaphoreType.DMA,
          pltpu.SemaphoreType.DMA,
          pltpu.SemaphoreType.DMA,
          pltpu.SemaphoreType.DMA,
          pltpu.SemaphoreType.DMA,
      ],
  )
  return kern(features.reshape(NP, DP), labels,
              centers.reshape(CP, DP)).reshape(C, D)


def kernel(features, labels, centers):
  return _run(features, labels.astype(jnp.int32), centers)


# speedup vs baseline: 1.1534x; 1.1534x over previous
import jax
import jax.numpy as jnp
from jax import lax
from jax.experimental import pallas as pl
from jax.experimental.pallas import tpu as pltpu
from jax.experimental.pallas import tpu_sc as plsc

N = 16384
D = 64
C = 100000
DP = 2 * D
NP = N // 2
NC = 2
NS = 16
NW = NC * NS
CPT = 3136
CPT_LAST = C - (NW - 1) * CPT
SLOTS = 512
LCHUNK = 512
GB = 96
CPROW = 160
LIST_CAP = N + 16
SENTINEL = 1 << 30


def _body(features, labels, centers, out,
          lab_buf, list_idx, list_crel, counts, slotmap, sums,
          wk_idx, wk_slot, wk_pair, bl_crel, bl_slot, rows,
          cp0, cp1, sem_g, sem_r0, sem_r1, sem_w0, sem_w1):
  wid = lax.axis_index("s") * NC + lax.axis_index("c")
  lo = wid * CPT
  iota16 = lax.iota(jnp.int32, 16)
  zeros16f = jnp.zeros((16,), jnp.float32)
  zeros16i = jnp.zeros((16,), jnp.int32)

  def compress_store(ref, x, m, base):
    mi = m.astype(jnp.int32)
    dest = base + plsc.cumsum(mi) - mi
    plsc.store_scatter(ref, [dest], x, mask=m)
    return plsc.all_reduce_population_count(m)[0]

  def z_list(i, _):
    list_idx[pl.ds(i * 16, 16)] = zeros16i
    list_crel[pl.ds(i * 16, 16)] = zeros16i
    return 0
  lax.fori_loop(0, LIST_CAP // 16, z_list, 0)
  def z_small(i, _):
    wk_idx[pl.ds(i * 16, 16)] = zeros16i
    wk_slot[pl.ds(i * 16, 16)] = zeros16i
    wk_pair[pl.ds(i * 16, 16)] = zeros16i
    return 0
  lax.fori_loop(0, (GB + 32) // 16, z_small, 0)

  def build_chunk(chunk, cur):
    pltpu.sync_copy(labels.at[pl.ds(chunk * LCHUNK, LCHUNK)], lab_buf)
    def group(g, cur):
      lab = lab_buf[pl.ds(g * 16, 16)]
      crel = lab - lo
      pos = iota16 + (chunk * LCHUNK + g * 16)
      m = (crel >= 0) & (crel < CPT)
      compress_store(list_idx, pos, m, cur)
      return cur + compress_store(list_crel, crel, m, cur)
    return lax.fori_loop(0, LCHUNK // 16, group, cur)
  cur = lax.fori_loop(0, N // LCHUNK, build_chunk, jnp.int32(0))

  def z_counts(z, _):
    counts[pl.ds(z * 16, 16)] = zeros16i
    return 0
  lax.fori_loop(0, CPT // 16, z_counts, 0)

  one_hot0 = (iota16 == 0).astype(jnp.int32)
  def add_count(i, _):
    c = list_crel[pl.ds(i, 16)][0]
    cnts = counts[pl.ds(c, 16)]
    counts[pl.ds(c, 16)] = cnts + one_hot0
    return 0
  lax.fori_loop(0, cur, add_count, 0)

  def slot_group(z, base):
    cv = counts[pl.ds(z * 16, 16)]
    mb = cv > 0
    m = mb.astype(jnp.int32)
    incl = plsc.cumsum(m)
    slotmap[pl.ds(z * 16, 16)] = base + incl - m
    return base + plsc.all_reduce_population_count(mb)[0]
  total_touched = lax.fori_loop(0, CPT // 16, slot_group, jnp.int32(0))

  def do_accum_flush(nvalid):
    pltpu.async_copy(features.at[wk_pair.at[pl.ds(0, GB)]], rows, sem_g).wait()
    def acc_row(j, _):
      @pl.when(j < nvalid)
      def _():
        s = wk_slot[pl.ds(j, 16)][0]
        par = wk_idx[pl.ds(j, 16)][0] & 1
        col0 = par * D
        for kk in range(D // 16):
          sv = sums[pl.ds(s * D + kk * 16, 16)]
          rv = rows[j, pl.ds(col0 + kk * 16, 16)]
          sums[pl.ds(s * D + kk * 16, 16)] = sv + rv
      return 0
    lax.fori_loop(0, GB, acc_row, 0)

  def blend_rows_in(buf, st, sz, state):
    end = st + sz
    def cond(s):
      _, peek = s
      return peek < end
    def body(s):
      bq, peek = s
      slot = bl_slot[pl.ds(bq, 16)][0]
      cnt = counts[pl.ds(peek, 16)][0]
      brel = peek - st
      cnt_v = jnp.full((16,), cnt, jnp.int32).astype(jnp.float32)
      w = 0.5 / cnt_v
      for kk in range(D // 16):
        cvv = buf[brel, pl.ds(kk * 16, 16)]
        svv = sums[pl.ds(slot * D + kk * 16, 16)]
        buf[brel, pl.ds(kk * 16, 16)] = 0.5 * cvv + w * svv
      bq = bq + 1
      return bq, bl_crel[pl.ds(bq, 16)][0]
    return lax.while_loop(cond, body, state)

  bufs = [cp0, cp1]
  rsems = [sem_r0, sem_r1]
  wsems = [sem_w0, sem_w1]

  def make_chunks(nrows):
    nfull = nrows // CPROW
    rem = nrows - nfull * CPROW
    chunks = [(i * CPROW, CPROW) for i in range(nfull)]
    if rem:
      chunks.append((nfull * CPROW, rem))
    return chunks

  def copy_blend_first(nrows):
    chunks = make_chunks(nrows)
    pend_w = [None, None]
    pend_r = [None, None]
    st0, sz0 = chunks[0]
    pend_r[0] = pltpu.async_copy(
        centers.at[pl.ds(lo + st0, sz0)], bufs[0].at[pl.ds(0, sz0)], rsems[0])
    state = (jnp.int32(0), bl_crel[pl.ds(0, 16)][0])
    for i, (st, sz) in enumerate(chunks):
      b = i % 2
      pend_r[b].wait()
      if i + 1 < len(chunks):
        nb = (i + 1) % 2
        if pend_w[nb] is not None:
          pend_w[nb].wait()
          pend_w[nb] = None
        nst, nsz = chunks[i + 1]
        pend_r[nb] = pltpu.async_copy(
            centers.at[pl.ds(lo + nst, nsz)], bufs[nb].at[pl.ds(0, nsz)],
            rsems[nb])
      state = blend_rows_in(bufs[b], st, sz, state)
      pend_w[b] = pltpu.async_copy(
          bufs[b].at[pl.ds(0, sz)], out.at[pl.ds(lo + st, sz)], wsems[b])
    for b in range(2):
      if pend_w[b] is not None:
        pend_w[b].wait()

  def copy_blend_rest(nrows):
    chunks = make_chunks(nrows)
    state = (jnp.int32(0), bl_crel[pl.ds(0, 16)][0])
    for st, sz in chunks:
      pltpu.sync_copy(out.at[pl.ds(lo + st, sz)], cp0.at[pl.ds(0, sz)])
      state = blend_rows_in(cp0, st, sz, state)
      pltpu.sync_copy(cp0.at[pl.ds(0, sz)], out.at[pl.ds(lo + st, sz)])

  def chunk_pass(k, _):
    slot_lo = k * SLOTS
    def z_sums(i, _):
      for kk in range(4):
        sums[pl.ds(i * 64 + kk * 16, 16)] = zeros16f
      return 0
    lax.fori_loop(0, SLOTS * D // 64, z_sums, 0)

    ngroups = (cur + 15) // 16
    def agroup(g, wcur):
      crel16 = list_crel[pl.ds(g * 16, 16)]
      idx16 = list_idx[pl.ds(g * 16, 16)]
      pos = iota16 + g * 16
      slot16 = plsc.load_gather(slotmap, [crel16])
      m = (pos < cur) & (slot16 >= slot_lo) & (slot16 < slot_lo + SLOTS)
      compress_store(wk_idx, idx16, m, wcur)
      compress_store(wk_pair, idx16 >> 1, m, wcur)
      wcur = wcur + compress_store(wk_slot, slot16 - slot_lo, m, wcur)
      @pl.when(wcur >= GB)
      def _():
        do_accum_flush(GB)
        wk_idx[pl.ds(0, 16)] = wk_idx[pl.ds(GB, 16)]
        wk_pair[pl.ds(0, 16)] = wk_pair[pl.ds(GB, 16)]
        wk_slot[pl.ds(0, 16)] = wk_slot[pl.ds(GB, 16)]
      return lax.select(wcur >= GB, wcur - GB, wcur)
    wcur = lax.fori_loop(0, ngroups, agroup, jnp.int32(0))
    @pl.when(wcur > 0)
    def _():
      do_accum_flush(wcur)

    sent16 = jnp.full((16,), SENTINEL, jnp.int32)
    def z_bl(i, _):
      bl_crel[pl.ds(i * 16, 16)] = sent16
      return 0
    lax.fori_loop(0, (SLOTS + 16) // 16, z_bl, 0)
    def bgroup(z, bcur):
      cv = counts[pl.ds(z * 16, 16)]
      crel16 = iota16 + z * 16
      slot16 = slotmap[pl.ds(z * 16, 16)]
      m = (cv > 0) & (slot16 >= slot_lo) & (slot16 < slot_lo + SLOTS)
      compress_store(bl_crel, crel16, m, bcur)
      return bcur + compress_store(bl_slot, slot16 - slot_lo, m, bcur)
    lax.fori_loop(0, CPT // 16, bgroup, jnp.int32(0))

    @pl.when(k == 0)
    def _():
      @pl.when(wid < NW - 1)
      def _():
        copy_blend_first(CPT)
      @pl.when(wid == NW - 1)
      def _():
        copy_blend_first(CPT_LAST)
    @pl.when(k > 0)
    def _():
      @pl.when(wid < NW - 1)
      def _():
        copy_blend_rest(CPT)
      @pl.when(wid == NW - 1)
      def _():
        copy_blend_rest(CPT_LAST)
    return 0

  nchunks = (total_touched + SLOTS - 1) // SLOTS
  nchunks = jnp.maximum(nchunks, 1)
  lax.fori_loop(0, nchunks, chunk_pass, 0)


@jax.jit
def _run(features, labels, centers):
  mesh = plsc.VectorSubcoreMesh(core_axis_name="c", subcore_axis_name="s",
                                num_cores=NC, num_subcores=NS)
  kern = pl.kernel(
      _body,
      out_type=jax.ShapeDtypeStruct((C, D), jnp.float32),
      mesh=mesh,
      compiler_params=pltpu.CompilerParams(needs_layout_passes=False),
      scratch_types=[
          pltpu.VMEM((LCHUNK,), jnp.int32),
          pltpu.VMEM((LIST_CAP,), jnp.int32),
          pltpu.VMEM((LIST_CAP,), jnp.int32),
          pltpu.VMEM((CPT + 16,), jnp.int32),
          pltpu.VMEM((CPT + 16,), jnp.int32),
          pltpu.VMEM((SLOTS * D,), jnp.float32),
          pltpu.VMEM((GB + 32,), jnp.int32),
          pltpu.VMEM((GB + 32,), jnp.int32),
          pltpu.VMEM((GB + 32,), jnp.int32),
          pltpu.VMEM((SLOTS + 32,), jnp.int32),
          pltpu.VMEM((SLOTS + 32,), jnp.int32),
          pltpu.VMEM((GB, DP), jnp.float32),
          pltpu.VMEM((CPROW, D), jnp.float32),
          pltpu.VMEM((CPROW, D), jnp.float32),
          pltpu.SemaphoreType.DMA,
          pltpu.SemaphoreType.DMA,
          pltpu.SemaphoreType.DMA,
          pltpu.SemaphoreType.DMA,
          pltpu.SemaphoreType.DMA,
      ],
  )
  return kern(features.reshape(NP, DP), labels, centers)


def kernel(features, labels, centers):
  return _run(features, labels.astype(jnp.int32), centers)


# speedup vs baseline: 1.1566x; 1.0028x over previous
import jax
import jax.numpy as jnp
from jax import lax
from jax.experimental import pallas as pl
from jax.experimental.pallas import tpu as pltpu
from jax.experimental.pallas import tpu_sc as plsc

N = 16384
D = 64
C = 100000
DP = 2 * D
NP = N // 2
NC = 2
NS = 16
NW = NC * NS
CPT = 3136
CPT_LAST = C - (NW - 1) * CPT
SLOTS = 512
LCHUNK = 512
GB = 96
CPROW = 160
LIST_CAP = N + 16
SENTINEL = 1 << 30


def _body(features, labels, centers, out,
          lab_buf, list_idx, list_crel, counts, slotmap, sums,
          wk_idx, wk_slot, wk_pair, bl_crel, bl_slot, rows,
          cp0, cp1, sem_g, sem_r0, sem_r1, sem_w0, sem_w1):
  wid = lax.axis_index("s") * NC + lax.axis_index("c")
  lo = wid * CPT
  iota16 = lax.iota(jnp.int32, 16)
  zeros16f = jnp.zeros((16,), jnp.float32)
  zeros16i = jnp.zeros((16,), jnp.int32)

  def compress_store(ref, x, m, base):
    mi = m.astype(jnp.int32)
    dest = base + plsc.cumsum(mi) - mi
    plsc.store_scatter(ref, [dest], x, mask=m)
    return plsc.all_reduce_population_count(m)[0]

  def z_list(i, _):
    list_idx[pl.ds(i * 16, 16)] = zeros16i
    list_crel[pl.ds(i * 16, 16)] = zeros16i
    return 0
  lax.fori_loop(0, LIST_CAP // 16, z_list, 0)
  def z_small(i, _):
    wk_idx[pl.ds(i * 16, 16)] = zeros16i
    wk_slot[pl.ds(i * 16, 16)] = zeros16i
    wk_pair[pl.ds(i * 16, 16)] = zeros16i
    return 0
  lax.fori_loop(0, (GB + 32) // 16, z_small, 0)

  def build_chunk(chunk, cur):
    pltpu.sync_copy(labels.at[pl.ds(chunk * LCHUNK, LCHUNK)], lab_buf)
    def group(g, cur):
      lab = lab_buf[pl.ds(g * 16, 16)]
      crel = lab - lo
      pos = iota16 + (chunk * LCHUNK + g * 16)
      m = (crel >= 0) & (crel < CPT)
      compress_store(list_idx, pos, m, cur)
      return cur + compress_store(list_crel, crel, m, cur)
    return lax.fori_loop(0, LCHUNK // 16, group, cur)
  cur = lax.fori_loop(0, N // LCHUNK, build_chunk, jnp.int32(0))

  def z_counts(z, _):
    counts[pl.ds(z * 16, 16)] = zeros16i
    return 0
  lax.fori_loop(0, CPT // 16, z_counts, 0)

  one_hot0 = (iota16 == 0).astype(jnp.int32)
  def add_count(i, _):
    c = list_crel[pl.ds(i, 16)][0]
    cnts = counts[pl.ds(c, 16)]
    counts[pl.ds(c, 16)] = cnts + one_hot0
    return 0
  lax.fori_loop(0, cur, add_count, 0)

  def slot_group(z, base):
    cv = counts[pl.ds(z * 16, 16)]
    mb = cv > 0
    m = mb.astype(jnp.int32)
    incl = plsc.cumsum(m)
    slotmap[pl.ds(z * 16, 16)] = base + incl - m
    return base + plsc.all_reduce_population_count(mb)[0]
  total_touched = lax.fori_loop(0, CPT // 16, slot_group, jnp.int32(0))

  def do_accum_flush(nvalid):
    pltpu.async_copy(features.at[wk_pair.at[pl.ds(0, GB)]], rows, sem_g).wait()
    def acc_row(j, _):
      @pl.when(j < nvalid)
      def _():
        s = wk_slot[pl.ds(j, 16)][0]
        par = wk_idx[pl.ds(j, 16)][0] & 1
        col0 = par * D
        for kk in range(D // 16):
          sv = sums[pl.ds(s * D + kk * 16, 16)]
          rv = rows[j, pl.ds(col0 + kk * 16, 16)]
          sums[pl.ds(s * D + kk * 16, 16)] = sv + rv
      return 0
    lax.fori_loop(0, GB, acc_row, 0)

  def blend_rows_in(buf, st, sz, state):
    end = st + sz
    def cond(s):
      _, peek = s
      return peek < end
    def body(s):
      bq, peek = s
      slot = bl_slot[pl.ds(bq, 16)][0]
      cnt = counts[pl.ds(peek, 16)][0]
      brel = peek - st
      cnt_v = jnp.full((16,), cnt, jnp.int32).astype(jnp.float32)
      w = 0.5 / cnt_v
      for kk in range(D // 16):
        cvv = buf[brel, pl.ds(kk * 16, 16)]
        svv = sums[pl.ds(slot * D + kk * 16, 16)]
        buf[brel, pl.ds(kk * 16, 16)] = 0.5 * cvv + w * svv
      bq = bq + 1
      return bq, bl_crel[pl.ds(bq, 16)][0]
    return lax.while_loop(cond, body, state)

  bufs = [cp0, cp1]
  rsems = [sem_r0, sem_r1]
  wsems = [sem_w0, sem_w1]

  def make_chunks(nrows):
    nfull = nrows // CPROW
    rem = nrows - nfull * CPROW
    chunks = [(i * CPROW, CPROW) for i in range(nfull)]
    if rem:
      chunks.append((nfull * CPROW, rem))
    return chunks

  def copy_blend_first(nrows):
    chunks = make_chunks(nrows)
    pend_w = [None, None]
    pend_r = [None, None]
    st0, sz0 = chunks[0]
    pend_r[0] = pltpu.async_copy(
        centers.at[pl.ds(lo + st0, sz0)], bufs[0].at[pl.ds(0, sz0)], rsems[0])
    state = (jnp.int32(0), bl_crel[pl.ds(0, 16)][0])
    for i, (st, sz) in enumerate(chunks):
      b = i % 2
      pend_r[b].wait()
      if i + 1 < len(chunks):
        nb = (i + 1) % 2
        if pend_w[nb] is not None:
          pend_w[nb].wait()
          pend_w[nb] = None
        nst, nsz = chunks[i + 1]
        pend_r[nb] = pltpu.async_copy(
            centers.at[pl.ds(lo + nst, nsz)], bufs[nb].at[pl.ds(0, nsz)],
            rsems[nb])
      state = blend_rows_in(bufs[b], st, sz, state)
      pend_w[b] = pltpu.async_copy(
          bufs[b].at[pl.ds(0, sz)], out.at[pl.ds(lo + st, sz)], wsems[b])
    for b in range(2):
      if pend_w[b] is not None:
        pend_w[b].wait()

  def copy_blend_rest(nrows):
    chunks = make_chunks(nrows)
    state = (jnp.int32(0), bl_crel[pl.ds(0, 16)][0])
    for st, sz in chunks:
      pltpu.sync_copy(out.at[pl.ds(lo + st, sz)], cp0.at[pl.ds(0, sz)])
      state = blend_rows_in(cp0, st, sz, state)
      pltpu.sync_copy(cp0.at[pl.ds(0, sz)], out.at[pl.ds(lo + st, sz)])

  def chunk_pass(k, _):
    slot_lo = k * SLOTS
    def z_sums(i, _):
      for kk in range(4):
        sums[pl.ds(i * 64 + kk * 16, 16)] = zeros16f
      return 0
    lax.fori_loop(0, SLOTS * D // 64, z_sums, 0)

    ngroups = (cur + 15) // 16
    def agroup(g, wcur):
      crel16 = list_crel[pl.ds(g * 16, 16)]
      idx16 = list_idx[pl.ds(g * 16, 16)]
      pos = iota16 + g * 16
      slot16 = plsc.load_gather(slotmap, [crel16])
      m = (pos < cur) & (slot16 >= slot_lo) & (slot16 < slot_lo + SLOTS)
      compress_store(wk_idx, idx16, m, wcur)
      compress_store(wk_pair, idx16 >> 1, m, wcur)
      wcur = wcur + compress_store(wk_slot, slot16 - slot_lo, m, wcur)
      @pl.when(wcur >= GB)
      def _():
        do_accum_flush(GB)
        wk_idx[pl.ds(0, 16)] = wk_idx[pl.ds(GB, 16)]
        wk_pair[pl.ds(0, 16)] = wk_pair[pl.ds(GB, 16)]
        wk_slot[pl.ds(0, 16)] = wk_slot[pl.ds(GB, 16)]
      return lax.select(wcur >= GB, wcur - GB, wcur)
    wcur = lax.fori_loop(0, ngroups, agroup, jnp.int32(0))
    @pl.when(wcur > 0)
    def _():
      do_accum_flush(wcur)

    sent16 = jnp.full((16,), SENTINEL, jnp.int32)
    def z_bl(i, _):
      bl_crel[pl.ds(i * 16, 16)] = sent16
      return 0
    lax.fori_loop(0, (SLOTS + 16) // 16, z_bl, 0)
    def bgroup(z, bcur):
      cv = counts[pl.ds(z * 16, 16)]
      crel16 = iota16 + z * 16
      slot16 = slotmap[pl.ds(z * 16, 16)]
      m = (cv > 0) & (slot16 >= slot_lo) & (slot16 < slot_lo + SLOTS)
      compress_store(bl_crel, crel16, m, bcur)
      return bcur + compress_store(bl_slot, slot16 - slot_lo, m, bcur)
    lax.fori_loop(0, CPT // 16, bgroup, jnp.int32(0))

    @pl.when(k == 0)
    def _():
      @pl.when(wid < NW - 1)
      def _():
        copy_blend_first(CPT)
      @pl.when(wid == NW - 1)
      def _():
        copy_blend_first(CPT_LAST)
    @pl.when(k > 0)
    def _():
      @pl.when(wid < NW - 1)
      def _():
        copy_blend_rest(CPT)
      @pl.when(wid == NW - 1)
      def _():
        copy_blend_rest(CPT_LAST)
    return 0

  nchunks = (total_touched + SLOTS - 1) // SLOTS
  nchunks = jnp.maximum(nchunks, 1)
  lax.fori_loop(0, nchunks, chunk_pass, 0)


@jax.jit
def _run(features, labels, centers):
  mesh = plsc.VectorSubcoreMesh(core_axis_name="c", subcore_axis_name="s",
                                num_cores=NC, num_subcores=NS)
  kern = pl.kernel(
      _body,
      out_type=jax.ShapeDtypeStruct((C, D), jnp.float32),
      mesh=mesh,
      compiler_params=pltpu.CompilerParams(needs_layout_passes=False,
                                           use_tc_tiling_on_sc=True),
      scratch_types=[
          pltpu.VMEM((LCHUNK,), jnp.int32),
          pltpu.VMEM((LIST_CAP,), jnp.int32),
          pltpu.VMEM((LIST_CAP,), jnp.int32),
          pltpu.VMEM((CPT + 16,), jnp.int32),
          pltpu.VMEM((CPT + 16,), jnp.int32),
          pltpu.VMEM((SLOTS * D,), jnp.float32),
          pltpu.VMEM((GB + 32,), jnp.int32),
          pltpu.VMEM((GB + 32,), jnp.int32),
          pltpu.VMEM((GB + 32,), jnp.int32),
          pltpu.VMEM((SLOTS + 32,), jnp.int32),
          pltpu.VMEM((SLOTS + 32,), jnp.int32),
          pltpu.VMEM((GB, DP), jnp.float32),
          pltpu.VMEM((CPROW, D), jnp.float32),
          pltpu.VMEM((CPROW, D), jnp.float32),
          pltpu.SemaphoreType.DMA,
          pltpu.SemaphoreType.DMA,
          pltpu.SemaphoreType.DMA,
          pltpu.SemaphoreType.DMA,
          pltpu.SemaphoreType.DMA,
      ],
  )
  return kern(features.reshape(NP, DP), labels, centers)


def kernel(features, labels, centers):
  return _run(features, labels.astype(jnp.int32), centers)


# speedup vs baseline: 1.5107x; 1.3061x over previous
import jax
import jax.numpy as jnp
from jax import lax
from jax.experimental import pallas as pl
from jax.experimental.pallas import tpu as pltpu
from jax.experimental.pallas import tpu_sc as plsc

N = 16384
D = 64
C = 100000
DP = 2 * D
NP = N // 2
NC = 2
NS = 16
NW = NC * NS
CPT = 3200
CPT_LAST = C - (NW - 1) * CPT
TAIL = 32
CPT_LAST_AL = CPT_LAST - TAIL
TAIL_REL = CPT_LAST_AL
SLOTS = 512
LCHUNK = 512
GB = 96
W = 256
LIST_CAP = N + 16
SENTINEL = 1 << 30


def _body(features, labels, centers_t, tail_rows, out_t, out_tail,
          lab_buf, list_idx, list_crel, counts, slotmap, sums,
          wk_idx, wk_slot, wk_pair, bl_crel, bl_slot, rows,
          cp0, cp1, tailbuf, sem_g, sem_r0, sem_r1, sem_w0, sem_w1):
  wid = lax.axis_index("s") * NC + lax.axis_index("c")
  lo = wid * CPT
  iota16 = lax.iota(jnp.int32, 16)
  zeros16f = jnp.zeros((16,), jnp.float32)
  zeros16i = jnp.zeros((16,), jnp.int32)

  def compress_store(ref, x, m, base):
    mi = m.astype(jnp.int32)
    dest = base + plsc.cumsum(mi) - mi
    plsc.store_scatter(ref, [dest], x, mask=m)
    return plsc.all_reduce_population_count(m)[0]

  def z_list(i, _):
    list_idx[pl.ds(i * 16, 16)] = zeros16i
    list_crel[pl.ds(i * 16, 16)] = zeros16i
    return 0
  lax.fori_loop(0, LIST_CAP // 16, z_list, 0)
  def z_small(i, _):
    wk_idx[pl.ds(i * 16, 16)] = zeros16i
    wk_slot[pl.ds(i * 16, 16)] = zeros16i
    wk_pair[pl.ds(i * 16, 16)] = zeros16i
    return 0
  lax.fori_loop(0, (GB + 32) // 16, z_small, 0)

  def build_chunk(chunk, cur):
    pltpu.sync_copy(labels.at[pl.ds(chunk * LCHUNK, LCHUNK)], lab_buf)
    def group(g, cur):
      lab = lab_buf[pl.ds(g * 16, 16)]
      crel = lab - lo
      pos = iota16 + (chunk * LCHUNK + g * 16)
      m = (crel >= 0) & (crel < CPT)
      compress_store(list_idx, pos, m, cur)
      return cur + compress_store(list_crel, crel, m, cur)
    return lax.fori_loop(0, LCHUNK // 16, group, cur)
  cur = lax.fori_loop(0, N // LCHUNK, build_chunk, jnp.int32(0))

  def z_counts(z, _):
    counts[pl.ds(z * 16, 16)] = zeros16i
    return 0
  lax.fori_loop(0, CPT // 16, z_counts, 0)

  one_hot0 = (iota16 == 0).astype(jnp.int32)
  def add_count(i, _):
    c = list_crel[pl.ds(i, 16)][0]
    cnts = counts[pl.ds(c, 16)]
    counts[pl.ds(c, 16)] = cnts + one_hot0
    return 0
  lax.fori_loop(0, cur, add_count, 0)

  def slot_group(z, base):
    cv = counts[pl.ds(z * 16, 16)]
    mb = cv > 0
    m = mb.astype(jnp.int32)
    incl = plsc.cumsum(m)
    slotmap[pl.ds(z * 16, 16)] = base + incl - m
    return base + plsc.all_reduce_population_count(mb)[0]
  total_touched = lax.fori_loop(0, CPT // 16, slot_group, jnp.int32(0))

  def do_accum_flush(nvalid):
    pltpu.async_copy(features.at[wk_pair.at[pl.ds(0, GB)]], rows, sem_g).wait()
    def acc_row(j, _):
      @pl.when(j < nvalid)
      def _():
        s = wk_slot[pl.ds(j, 16)][0]
        par = wk_idx[pl.ds(j, 16)][0] & 1
        col0 = par * D
        for kk in range(D // 16):
          sv = sums[pl.ds(s * D + kk * 16, 16)]
          rv = rows[j, pl.ds(col0 + kk * 16, 16)]
          sums[pl.ds(s * D + kk * 16, 16)] = sv + rv
      return 0
    lax.fori_loop(0, GB, acc_row, 0)

  def blend_cols_in(buf, st, sz, state):
    end = st + sz
    def cond(s):
      _, peek = s
      return peek < end
    def body(s):
      bq, peek = s
      slot = bl_slot[pl.ds(bq, 16)][0]
      cnt = counts[pl.ds(peek, 16)][0]
      col16 = jnp.full((16,), peek - st, jnp.int32)
      cnt_v = jnp.full((16,), cnt, jnp.int32).astype(jnp.float32)
      w = 0.5 / cnt_v
      for kk in range(D // 16):
        d16 = iota16 + kk * 16
        cvv = plsc.load_gather(buf, [d16, col16])
        svv = sums[pl.ds(slot * D + kk * 16, 16)]
        plsc.store_scatter(buf, [d16, col16], 0.5 * cvv + w * svv)
      bq = bq + 1
      return bq, bl_crel[pl.ds(bq, 16)][0]
    return lax.while_loop(cond, body, state)

  bufs = [cp0, cp1]
  rsems = [sem_r0, sem_r1]
  wsems = [sem_w0, sem_w1]

  def make_chunks(ncols):
    nfull = ncols // W
    rem = ncols - nfull * W
    chunks = [(i * W, W) for i in range(nfull)]
    if rem:
      chunks.append((nfull * W, rem))
    return chunks

  def copy_blend_first(ncols):
    chunks = make_chunks(ncols)
    pend_w = [None, None]
    pend_r = [None, None]
    st0, sz0 = chunks[0]
    pend_r[0] = pltpu.async_copy(
        centers_t.at[:, pl.ds(lo + st0, sz0)], bufs[0].at[:, pl.ds(0, sz0)],
        rsems[0])
    state = (jnp.int32(0), bl_crel[pl.ds(0, 16)][0])
    for i, (st, sz) in enumerate(chunks):
      b = i % 2
      pend_r[b].wait()
      if i + 1 < len(chunks):
        nb = (i + 1) % 2
        if pend_w[nb] is not None:
          pend_w[nb].wait()
          pend_w[nb] = None
        nst, nsz = chunks[i + 1]
        pend_r[nb] = pltpu.async_copy(
            centers_t.at[:, pl.ds(lo + nst, nsz)],
            bufs[nb].at[:, pl.ds(0, nsz)], rsems[nb])
      state = blend_cols_in(bufs[b], st, sz, state)
      pend_w[b] = pltpu.async_copy(
          bufs[b].at[:, pl.ds(0, sz)], out_t.at[:, pl.ds(lo + st, sz)],
          wsems[b])
    for b in range(2):
      if pend_w[b] is not None:
        pend_w[b].wait()

  def copy_blend_rest(ncols):
    chunks = make_chunks(ncols)
    state = (jnp.int32(0), bl_crel[pl.ds(0, 16)][0])
    for st, sz in chunks:
      pltpu.sync_copy(out_t.at[:, pl.ds(lo + st, sz)], cp0.at[:, pl.ds(0, sz)])
      state = blend_cols_in(cp0, st, sz, state)
      pltpu.sync_copy(cp0.at[:, pl.ds(0, sz)], out_t.at[:, pl.ds(lo + st, sz)])

  def tail_pass(slot_lo, first):
    if first:
      pltpu.sync_copy(tail_rows, tailbuf)
    else:
      pltpu.sync_copy(out_tail, tailbuf)
    for c in range(TAIL):
      crel = TAIL_REL + c
      cnt = counts[pl.ds(crel, 16)][0]
      slot = slotmap[pl.ds(crel, 16)][0]
      @pl.when((cnt > 0) & (slot >= slot_lo) & (slot < slot_lo + SLOTS))
      def _():
        srel = slot - slot_lo
        cnt_v = jnp.full((16,), cnt, jnp.int32).astype(jnp.float32)
        w = 0.5 / cnt_v
        for kk in range(D // 16):
          vv = tailbuf[c, pl.ds(kk * 16, 16)]
          svv = sums[pl.ds(srel * D + kk * 16, 16)]
          tailbuf[c, pl.ds(kk * 16, 16)] = 0.5 * vv + w * svv
    pltpu.sync_copy(tailbuf, out_tail)

  def chunk_pass(k, _):
    slot_lo = k * SLOTS
    def z_sums(i, _):
      for kk in range(4):
        sums[pl.ds(i * 64 + kk * 16, 16)] = zeros16f
      return 0
    lax.fori_loop(0, SLOTS * D // 64, z_sums, 0)

    ngroups = (cur + 15) // 16
    def agroup(g, wcur):
      crel16 = list_crel[pl.ds(g * 16, 16)]
      idx16 = list_idx[pl.ds(g * 16, 16)]
      pos = iota16 + g * 16
      slot16 = plsc.load_gather(slotmap, [crel16])
      m = (pos < cur) & (slot16 >= slot_lo) & (slot16 < slot_lo + SLOTS)
      compress_store(wk_idx, idx16, m, wcur)
      compress_store(wk_pair, idx16 >> 1, m, wcur)
      wcur = wcur + compress_store(wk_slot, slot16 - slot_lo, m, wcur)
      @pl.when(wcur >= GB)
      def _():
        do_accum_flush(GB)
        wk_idx[pl.ds(0, 16)] = wk_idx[pl.ds(GB, 16)]
        wk_pair[pl.ds(0, 16)] = wk_pair[pl.ds(GB, 16)]
        wk_slot[pl.ds(0, 16)] = wk_slot[pl.ds(GB, 16)]
      return lax.select(wcur >= GB, wcur - GB, wcur)
    wcur = lax.fori_loop(0, ngroups, agroup, jnp.int32(0))
    @pl.when(wcur > 0)
    def _():
      do_accum_flush(wcur)

    sent16 = jnp.full((16,), SENTINEL, jnp.int32)
    def z_bl(i, _):
      bl_crel[pl.ds(i * 16, 16)] = sent16
      return 0
    lax.fori_loop(0, (SLOTS + 16) // 16, z_bl, 0)
    def bgroup(z, bcur):
      cv = counts[pl.ds(z * 16, 16)]
      crel16 = iota16 + z * 16
      slot16 = slotmap[pl.ds(z * 16, 16)]
      m = (cv > 0) & (slot16 >= slot_lo) & (slot16 < slot_lo + SLOTS)
      compress_store(bl_crel, crel16, m, bcur)
      return bcur + compress_store(bl_slot, slot16 - slot_lo, m, bcur)
    lax.fori_loop(0, CPT // 16, bgroup, jnp.int32(0))

    @pl.when(k == 0)
    def _():
      @pl.when(wid < NW - 1)
      def _():
        copy_blend_first(CPT)
      @pl.when(wid == NW - 1)
      def _():
        copy_blend_first(CPT_LAST_AL)
        tail_pass(slot_lo, first=True)
    @pl.when(k > 0)
    def _():
      @pl.when(wid < NW - 1)
      def _():
        copy_blend_rest(CPT)
      @pl.when(wid == NW - 1)
      def _():
        copy_blend_rest(CPT_LAST_AL)
        tail_pass(slot_lo, first=False)
    return 0

  nchunks = (total_touched + SLOTS - 1) // SLOTS
  nchunks = jnp.maximum(nchunks, 1)
  lax.fori_loop(0, nchunks, chunk_pass, 0)


@jax.jit
def _run(features, labels, centers):
  mesh = plsc.VectorSubcoreMesh(core_axis_name="c", subcore_axis_name="s",
                                num_cores=NC, num_subcores=NS)
  kern = pl.kernel(
      _body,
      out_type=(jax.ShapeDtypeStruct((D, C), jnp.float32),
                jax.ShapeDtypeStruct((TAIL, D), jnp.float32)),
      mesh=mesh,
      compiler_params=pltpu.CompilerParams(needs_layout_passes=False,
                                           use_tc_tiling_on_sc=True),
      scratch_types=[
          pltpu.VMEM((LCHUNK,), jnp.int32),
          pltpu.VMEM((LIST_CAP,), jnp.int32),
          pltpu.VMEM((LIST_CAP,), jnp.int32),
          pltpu.VMEM((CPT + 16,), jnp.int32),
          pltpu.VMEM((CPT + 16,), jnp.int32),
          pltpu.VMEM((SLOTS * D,), jnp.float32),
          pltpu.VMEM((GB + 32,), jnp.int32),
          pltpu.VMEM((GB + 32,), jnp.int32),
          pltpu.VMEM((GB + 32,), jnp.int32),
          pltpu.VMEM((SLOTS + 32,), jnp.int32),
          pltpu.VMEM((SLOTS + 32,), jnp.int32),
          pltpu.VMEM((GB, DP), jnp.float32),
          pltpu.VMEM((D, W), jnp.float32),
          pltpu.VMEM((D, W), jnp.float32),
          pltpu.VMEM((TAIL, D), jnp.float32),
          pltpu.SemaphoreType.DMA,
          pltpu.SemaphoreType.DMA,
          pltpu.SemaphoreType.DMA,
          pltpu.SemaphoreType.DMA,
          pltpu.SemaphoreType.DMA,
      ],
  )
  tail_rows = lax.slice(centers, (C - TAIL, 0), (C, D))
  out_t, out_tail = kern(features.reshape(NP, DP), labels, centers.T,
                         tail_rows)
  out_t = lax.dynamic_update_slice(out_t, out_tail.T, (0, C - TAIL))
  return out_t.T


def kernel(features, labels, centers):
  return _run(features, labels.astype(jnp.int32), centers)


# speedup vs baseline: 1.5435x; 1.0217x over previous
import jax
import jax.numpy as jnp
from jax import lax
from jax.experimental import pallas as pl
from jax.experimental.pallas import tpu as pltpu
from jax.experimental.pallas import tpu_sc as plsc

N = 16384
D = 64
C = 100000
DP = 2 * D
NP = N // 2
NC = 2
NS = 16
NW = NC * NS
CPT = 3200
CPT_LAST = C - (NW - 1) * CPT
TAIL = 32
CPT_LAST_AL = CPT_LAST - TAIL
TAIL_REL = CPT_LAST_AL
SLOTS = 512
LCHUNK = 512
GB = 96
W = 256
LIST_CAP = N + 16
SENTINEL = 1 << 30


def _body(features, labels, centers_t, tail_rows, out_t, out_tail,
          lab_buf, list_idx, list_crel, counts, slotmap, sums,
          wk_idx, wk_slot, wk_pair, bl_crel, bl_slot, rows,
          cp0, cp1, tailbuf, sem_g, sem_r0, sem_r1, sem_w0, sem_w1):
  wid = lax.axis_index("s") * NC + lax.axis_index("c")
  lo = wid * CPT
  iota16 = lax.iota(jnp.int32, 16)
  zeros16f = jnp.zeros((16,), jnp.float32)
  zeros16i = jnp.zeros((16,), jnp.int32)

  def compress_store(ref, x, m, base):
    mi = m.astype(jnp.int32)
    dest = base + plsc.cumsum(mi) - mi
    plsc.store_scatter(ref, [dest], x, mask=m)
    return plsc.all_reduce_population_count(m)[0]

  def compress_store_multi(pairs, m, base):
    mi = m.astype(jnp.int32)
    dest = base + plsc.cumsum(mi) - mi
    for ref, x in pairs:
      plsc.store_scatter(ref, [dest], x, mask=m)
    return base + plsc.all_reduce_population_count(m)[0]

  def z_list(i, _):
    list_idx[pl.ds(i * 16, 16)] = zeros16i
    list_crel[pl.ds(i * 16, 16)] = zeros16i
    return 0
  lax.fori_loop(0, LIST_CAP // 16, z_list, 0, unroll=8)
  def z_small(i, _):
    wk_idx[pl.ds(i * 16, 16)] = zeros16i
    wk_slot[pl.ds(i * 16, 16)] = zeros16i
    wk_pair[pl.ds(i * 16, 16)] = zeros16i
    return 0
  lax.fori_loop(0, (GB + 32) // 16, z_small, 0)

  def build_chunk(chunk, cur):
    pltpu.sync_copy(labels.at[pl.ds(chunk * LCHUNK, LCHUNK)], lab_buf)
    def group(g, cur):
      lab = lab_buf[pl.ds(g * 16, 16)]
      crel = lab - lo
      pos = iota16 + (chunk * LCHUNK + g * 16)
      m = (crel >= 0) & (crel < CPT)
      return compress_store_multi(
          [(list_idx, pos), (list_crel, crel)], m, cur)
    return lax.fori_loop(0, LCHUNK // 16, group, cur, unroll=4)
  cur = lax.fori_loop(0, N // LCHUNK, build_chunk, jnp.int32(0))

  def z_counts(z, _):
    counts[pl.ds(z * 16, 16)] = zeros16i
    return 0
  lax.fori_loop(0, CPT // 16, z_counts, 0, unroll=8)

  one_hot0 = (iota16 == 0).astype(jnp.int32)
  def add_count(i, _):
    c = list_crel[pl.ds(i, 16)][0]
    cnts = counts[pl.ds(c, 16)]
    counts[pl.ds(c, 16)] = cnts + one_hot0
    return 0
  lax.fori_loop(0, cur, add_count, 0)

  def slot_group(z, base):
    cv = counts[pl.ds(z * 16, 16)]
    mb = cv > 0
    m = mb.astype(jnp.int32)
    incl = plsc.cumsum(m)
    slotmap[pl.ds(z * 16, 16)] = base + incl - m
    return base + plsc.all_reduce_population_count(mb)[0]
  total_touched = lax.fori_loop(0, CPT // 16, slot_group, jnp.int32(0), unroll=4)

  def do_accum_flush(nvalid):
    pltpu.async_copy(features.at[wk_pair.at[pl.ds(0, GB)]], rows, sem_g).wait()
    def acc_row(j, _):
      @pl.when(j < nvalid)
      def _():
        s = wk_slot[pl.ds(j, 16)][0]
        par = wk_idx[pl.ds(j, 16)][0] & 1
        col0 = par * D
        for kk in range(D // 16):
          sv = sums[pl.ds(s * D + kk * 16, 16)]
          rv = rows[j, pl.ds(col0 + kk * 16, 16)]
          sums[pl.ds(s * D + kk * 16, 16)] = sv + rv
      return 0
    lax.fori_loop(0, GB, acc_row, 0, unroll=2)

  def blend_cols_in(buf, st, sz, state):
    end = st + sz
    def cond(s):
      _, peek = s
      return peek < end
    def body(s):
      bq, peek = s
      slot = bl_slot[pl.ds(bq, 16)][0]
      cnt = counts[pl.ds(peek, 16)][0]
      col16 = jnp.full((16,), peek - st, jnp.int32)
      cnt_v = jnp.full((16,), cnt, jnp.int32).astype(jnp.float32)
      w = 0.5 / cnt_v
      for kk in range(D // 16):
        d16 = iota16 + kk * 16
        cvv = plsc.load_gather(buf, [d16, col16])
        svv = sums[pl.ds(slot * D + kk * 16, 16)]
        plsc.store_scatter(buf, [d16, col16], 0.5 * cvv + w * svv)
      bq = bq + 1
      return bq, bl_crel[pl.ds(bq, 16)][0]
    return lax.while_loop(cond, body, state)

  bufs = [cp0, cp1]
  rsems = [sem_r0, sem_r1]
  wsems = [sem_w0, sem_w1]

  def make_chunks(ncols):
    nfull = ncols // W
    rem = ncols - nfull * W
    chunks = [(i * W, W) for i in range(nfull)]
    if rem:
      chunks.append((nfull * W, rem))
    return chunks

  def copy_blend_first(ncols):
    chunks = make_chunks(ncols)
    pend_w = [None, None]
    pend_r = [None, None]
    st0, sz0 = chunks[0]
    pend_r[0] = pltpu.async_copy(
        centers_t.at[:, pl.ds(lo + st0, sz0)], bufs[0].at[:, pl.ds(0, sz0)],
        rsems[0])
    state = (jnp.int32(0), bl_crel[pl.ds(0, 16)][0])
    for i, (st, sz) in enumerate(chunks):
      b = i % 2
      pend_r[b].wait()
      if i + 1 < len(chunks):
        nb = (i + 1) % 2
        if pend_w[nb] is not None:
          pend_w[nb].wait()
          pend_w[nb] = None
        nst, nsz = chunks[i + 1]
        pend_r[nb] = pltpu.async_copy(
            centers_t.at[:, pl.ds(lo + nst, nsz)],
            bufs[nb].at[:, pl.ds(0, nsz)], rsems[nb])
      state = blend_cols_in(bufs[b], st, sz, state)
      pend_w[b] = pltpu.async_copy(
          bufs[b].at[:, pl.ds(0, sz)], out_t.at[:, pl.ds(lo + st, sz)],
          wsems[b])
    for b in range(2):
      if pend_w[b] is not None:
        pend_w[b].wait()

  def copy_blend_rest(ncols):
    chunks = make_chunks(ncols)
    state = (jnp.int32(0), bl_crel[pl.ds(0, 16)][0])
    for st, sz in chunks:
      pltpu.sync_copy(out_t.at[:, pl.ds(lo + st, sz)], cp0.at[:, pl.ds(0, sz)])
      state = blend_cols_in(cp0, st, sz, state)
      pltpu.sync_copy(cp0.at[:, pl.ds(0, sz)], out_t.at[:, pl.ds(lo + st, sz)])

  def tail_pass(slot_lo, first):
    if first:
      pltpu.sync_copy(tail_rows, tailbuf)
    else:
      pltpu.sync_copy(out_tail, tailbuf)
    for c in range(TAIL):
      crel = TAIL_REL + c
      cnt = counts[pl.ds(crel, 16)][0]
      slot = slotmap[pl.ds(crel, 16)][0]
      @pl.when((cnt > 0) & (slot >= slot_lo) & (slot < slot_lo + SLOTS))
      def _():
        srel = slot - slot_lo
        cnt_v = jnp.full((16,), cnt, jnp.int32).astype(jnp.float32)
        w = 0.5 / cnt_v
        for kk in range(D // 16):
          vv = tailbuf[c, pl.ds(kk * 16, 16)]
          svv = sums[pl.ds(srel * D + kk * 16, 16)]
          tailbuf[c, pl.ds(kk * 16, 16)] = 0.5 * vv + w * svv
    pltpu.sync_copy(tailbuf, out_tail)

  def chunk_pass(k, _):
    slot_lo = k * SLOTS
    def z_sums(i, _):
      for kk in range(4):
        sums[pl.ds(i * 64 + kk * 16, 16)] = zeros16f
      return 0
    lax.fori_loop(0, SLOTS * D // 64, z_sums, 0, unroll=8)

    ngroups = (cur + 15) // 16
    def agroup(g, wcur):
      crel16 = list_crel[pl.ds(g * 16, 16)]
      idx16 = list_idx[pl.ds(g * 16, 16)]
      pos = iota16 + g * 16
      slot16 = plsc.load_gather(slotmap, [crel16])
      m = (pos < cur) & (slot16 >= slot_lo) & (slot16 < slot_lo + SLOTS)
      wcur = compress_store_multi(
          [(wk_idx, idx16), (wk_pair, idx16 >> 1),
           (wk_slot, slot16 - slot_lo)], m, wcur)
      @pl.when(wcur >= GB)
      def _():
        do_accum_flush(GB)
        wk_idx[pl.ds(0, 16)] = wk_idx[pl.ds(GB, 16)]
        wk_pair[pl.ds(0, 16)] = wk_pair[pl.ds(GB, 16)]
        wk_slot[pl.ds(0, 16)] = wk_slot[pl.ds(GB, 16)]
      return lax.select(wcur >= GB, wcur - GB, wcur)
    wcur = lax.fori_loop(0, ngroups, agroup, jnp.int32(0))
    @pl.when(wcur > 0)
    def _():
      do_accum_flush(wcur)

    sent16 = jnp.full((16,), SENTINEL, jnp.int32)
    def z_bl(i, _):
      bl_crel[pl.ds(i * 16, 16)] = sent16
      return 0
    lax.fori_loop(0, (SLOTS + 16) // 16, z_bl, 0)
    def bgroup(z, bcur):
      cv = counts[pl.ds(z * 16, 16)]
      crel16 = iota16 + z * 16
      slot16 = slotmap[pl.ds(z * 16, 16)]
      m = (cv > 0) & (slot16 >= slot_lo) & (slot16 < slot_lo + SLOTS)
      return compress_store_multi(
          [(bl_crel, crel16), (bl_slot, slot16 - slot_lo)], m, bcur)
    lax.fori_loop(0, CPT // 16, bgroup, jnp.int32(0), unroll=4)

    @pl.when(k == 0)
    def _():
      @pl.when(wid < NW - 1)
      def _():
        copy_blend_first(CPT)
      @pl.when(wid == NW - 1)
      def _():
        copy_blend_first(CPT_LAST_AL)
        tail_pass(slot_lo, first=True)
    @pl.when(k > 0)
    def _():
      @pl.when(wid < NW - 1)
      def _():
        copy_blend_rest(CPT)
      @pl.when(wid == NW - 1)
      def _():
        copy_blend_rest(CPT_LAST_AL)
        tail_pass(slot_lo, first=False)
    return 0

  nchunks = (total_touched + SLOTS - 1) // SLOTS
  nchunks = jnp.maximum(nchunks, 1)
  lax.fori_loop(0, nchunks, chunk_pass, 0)


@jax.jit
def _run(features, labels, centers):
  mesh = plsc.VectorSubcoreMesh(core_axis_name="c", subcore_axis_name="s",
                                num_cores=NC, num_subcores=NS)
  kern = pl.kernel(
      _body,
      out_type=(jax.ShapeDtypeStruct((D, C), jnp.float32),
                jax.ShapeDtypeStruct((TAIL, D), jnp.float32)),
      mesh=mesh,
      compiler_params=pltpu.CompilerParams(needs_layout_passes=False,
                                           use_tc_tiling_on_sc=True),
      scratch_types=[
          pltpu.VMEM((LCHUNK,), jnp.int32),
          pltpu.VMEM((LIST_CAP,), jnp.int32),
          pltpu.VMEM((LIST_CAP,), jnp.int32),
          pltpu.VMEM((CPT + 16,), jnp.int32),
          pltpu.VMEM((CPT + 16,), jnp.int32),
          pltpu.VMEM((SLOTS * D,), jnp.float32),
          pltpu.VMEM((GB + 32,), jnp.int32),
          pltpu.VMEM((GB + 32,), jnp.int32),
          pltpu.VMEM((GB + 32,), jnp.int32),
          pltpu.VMEM((SLOTS + 32,), jnp.int32),
          pltpu.VMEM((SLOTS + 32,), jnp.int32),
          pltpu.VMEM((GB, DP), jnp.float32),
          pltpu.VMEM((D, W), jnp.float32),
          pltpu.VMEM((D, W), jnp.float32),
          pltpu.VMEM((TAIL, D), jnp.float32),
          pltpu.SemaphoreType.DMA,
          pltpu.SemaphoreType.DMA,
          pltpu.SemaphoreType.DMA,
          pltpu.SemaphoreType.DMA,
          pltpu.SemaphoreType.DMA,
      ],
  )
  tail_rows = lax.slice(centers, (C - TAIL, 0), (C, D))
  out_t, out_tail = kern(features.reshape(NP, DP), labels, centers.T,
                         tail_rows)
  out_t = lax.dynamic_update_slice(out_t, out_tail.T, (0, C - TAIL))
  return out_t.T


def kernel(features, labels, centers):
  return _run(features, labels.astype(jnp.int32), centers)
